# baseline (speedup 1.0000x reference)
import functools

import jax
import jax.numpy as jnp
from jax import lax
from jax.experimental import pallas as pl
from jax.experimental.pallas import tpu as pltpu
from jax.experimental.pallas import tpu_sc as plsc

NS = 16
BLK = 128
G = 3


def _sc_body(n_acc, ng, xflat, colb, rowb, cola, rowa, zeros, dummy,
             out, acc1, acc2, cv0, wv0, cv1, wv1, rv0, rv1,
             gsem0, gsem1, ssem0, ssem1):
  c = lax.axis_index("c")
  s = lax.axis_index("s")
  stripe = n_acc // NS
  t_pairs = ng // 2

  def wait(sem):
    pltpu.make_async_copy(dummy, rv0, sem).wait()

  def do_pass(col_at, row_hbm, src, acc, dst_hbm):
    pltpu.sync_copy(zeros, acc.at[pl.ds(s * stripe, stripe)])
    plsc.subcore_barrier()
    base = s * ng

    def load(gidx, cv, wv):
      pltpu.sync_copy(col_at(gidx), cv)
      pltpu.sync_copy(row_hbm.at[gidx], wv)

    def gather(cv, rv, sem):
      for g in range(G):
        pltpu.async_copy(src.at[cv.at[g]], rv.at[g], sem)

    def scatter(rv, wv, sem):
      for g in range(G):
        pltpu.async_copy(rv.at[g], acc.at[wv.at[g]], sem, add=True)

    load(base, cv0, wv0)
    gather(cv0, rv0, gsem0)
    load(base + 1, cv1, wv1)
    gather(cv1, rv1, gsem1)
    wait(gsem0)
    scatter(rv0, wv0, ssem0)
    wait(ssem0)
    load(base + 2, cv0, wv0)
    gather(cv0, rv0, gsem0)
    wait(gsem1)
    scatter(rv1, wv1, ssem1)

    def body(t, carry):
      g = base + 2 * t
      wait(ssem1)
      load(g + 1, cv1, wv1)
      gather(cv1, rv1, gsem1)
      wait(gsem0)
      scatter(rv0, wv0, ssem0)
      wait(ssem0)

      @pl.when(t < t_pairs - 1)
      def _():
        load(g + 2, cv0, wv0)
        gather(cv0, rv0, gsem0)

      wait(gsem1)
      scatter(rv1, wv1, ssem1)
      return carry

    lax.fori_loop(1, t_pairs, body, 0, unroll=False)
    wait(ssem1)
    plsc.subcore_barrier()
    if dst_hbm is not None:
      pltpu.sync_copy(acc.at[pl.ds(s * stripe, stripe)],
                      dst_hbm.at[pl.ds(c * n_acc + s * stripe, stripe)])

  do_pass(lambda g: colb.at[c, g], rowb, xflat, acc1, None)
  do_pass(lambda g: cola.at[g], rowa, acc1, acc2, out)


def kernel(x, edge_index_a, edge_index_b):
  n, d = x.shape
  half = d // 2
  e = edge_index_a.shape[1]

  blocks = -(-e // BLK)
  ng = -(-blocks // (NS * G))
  ng = ng + (ng % 2)
  e_pad = NS * ng * G * BLK
  pad = e_pad - e

  n_acc = -(-(n + 2) // NS) * NS

  padrows = jnp.zeros((n_acc - n, half), jnp.float32)
  xflat = jnp.concatenate(
      [x[:, :half], padrows, x[:, half:], padrows], axis=0)

  def prep_cols(idx):
    p = jnp.concatenate([idx, jnp.zeros((pad,), jnp.int32)])
    both = jnp.stack([p, p + n_acc])
    return both.reshape(2, NS * ng, G, BLK)

  def prep_rows(idx, fill):
    p = jnp.concatenate([idx, jnp.full((pad,), fill, jnp.int32)])
    return p.reshape(NS * ng, G, BLK)

  colb = prep_cols(edge_index_b[1])
  rowb = prep_rows(edge_index_b[0], n)
  cola = prep_rows(edge_index_a[1], n + 1)
  rowa = prep_rows(edge_index_a[0], n)

  zeros = jnp.zeros((n_acc // NS, half), jnp.float32)
  dummy = jnp.zeros((G, BLK, half), jnp.float32)

  mesh = plsc.VectorSubcoreMesh(core_axis_name="c", subcore_axis_name="s")
  fn = pl.kernel(
      functools.partial(_sc_body, n_acc, ng),
      out_type=jax.ShapeDtypeStruct((2 * n_acc, half), jnp.float32),
      mesh=mesh,
      scratch_types=[
          pltpu.VMEM_SHARED((n_acc, half), jnp.float32),
          pltpu.VMEM_SHARED((n_acc, half), jnp.float32),
          pltpu.VMEM((G, BLK), jnp.int32),
          pltpu.VMEM((G, BLK), jnp.int32),
          pltpu.VMEM((G, BLK), jnp.int32),
          pltpu.VMEM((G, BLK), jnp.int32),
          pltpu.VMEM((G, BLK, half), jnp.float32),
          pltpu.VMEM((G, BLK, half), jnp.float32),
          pltpu.SemaphoreType.DMA,
          pltpu.SemaphoreType.DMA,
          pltpu.SemaphoreType.DMA,
          pltpu.SemaphoreType.DMA,
      ],
      compiler_params=pltpu.CompilerParams(use_tc_tiling_on_sc=False),
  )
  outflat = fn(xflat, colb, rowb, cola, rowa, zeros, dummy)
  return jnp.concatenate([outflat[:n], outflat[n_acc:n_acc + n]], axis=1)


# speedup vs baseline: 1.0874x; 1.0874x over previous
import functools

import jax
import jax.numpy as jnp
from jax import lax
from jax.experimental import pallas as pl
from jax.experimental.pallas import tpu as pltpu
from jax.experimental.pallas import tpu_sc as plsc

NS = 16
BLK = 128
G = 2


def _sc_body(n_acc, ng, xflat, idxb, idxa, zeros, dummy,
             out, acc1, acc2, iv0, iv1, rv0, rv1,
             gsem0, gsem1, ssem0, ssem1):
  c = lax.axis_index("c")
  s = lax.axis_index("s")
  stripe = n_acc // NS
  t_pairs = ng // 2

  def wait(sem):
    pltpu.make_async_copy(dummy, rv0, sem).wait()

  def do_pass(idx_at, src, acc, dst_hbm, zero_accs):
    base = s * ng

    def load(gidx, iv):
      pltpu.sync_copy(idx_at(gidx), iv)

    def gather(iv, rv, sem):
      for g in range(G):
        pltpu.async_copy(src.at[iv.at[0, g]], rv.at[g], sem)

    def scatter(rv, iv, sem):
      for g in range(G):
        pltpu.async_copy(rv.at[g], acc.at[iv.at[1, g]], sem, add=True)

    load(base, iv0)
    gather(iv0, rv0, gsem0)
    load(base + 1, iv1)
    gather(iv1, rv1, gsem1)
    for z in zero_accs:
      pltpu.sync_copy(zeros, z.at[pl.ds(s * stripe, stripe)])
    if zero_accs:
      plsc.subcore_barrier()
    wait(gsem0)
    scatter(rv0, iv0, ssem0)
    wait(ssem0)
    load(base + 2, iv0)
    gather(iv0, rv0, gsem0)
    wait(gsem1)
    scatter(rv1, iv1, ssem1)

    def body(t, carry):
      g = base + 2 * t
      wait(ssem1)
      load(g + 1, iv1)
      gather(iv1, rv1, gsem1)
      wait(gsem0)
      scatter(rv0, iv0, ssem0)
      wait(ssem0)

      @pl.when(t < t_pairs - 1)
      def _():
        load(g + 2, iv0)
        gather(iv0, rv0, gsem0)

      wait(gsem1)
      scatter(rv1, iv1, ssem1)
      return carry

    lax.fori_loop(1, t_pairs, body, 0, unroll=False)
    wait(ssem1)
    plsc.subcore_barrier()
    if dst_hbm is not None:
      pltpu.sync_copy(acc.at[pl.ds(s * stripe, stripe)],
                      dst_hbm.at[pl.ds(c * n_acc + s * stripe, stripe)])

  do_pass(lambda g: idxb.at[c, g], xflat, acc1, None, (acc1, acc2))
  do_pass(lambda g: idxa.at[g], acc1, acc2, out, ())


def kernel(x, edge_index_a, edge_index_b):
  n, d = x.shape
  half = d // 2
  e = edge_index_a.shape[1]

  blocks = -(-e // BLK)
  ng = -(-blocks // (NS * G))
  ng = ng + (ng % 2)
  e_pad = NS * ng * G * BLK
  pad = e_pad - e

  n_acc = -(-n // (8 * NS)) * 8 * NS

  padrows = jnp.zeros((n_acc - n, half), jnp.float32)
  xflat = jnp.concatenate(
      [x[:, :half], padrows, x[:, half:], padrows], axis=0)

  def prep(cols, rows, col_fill, bias):
    cp = jnp.concatenate([cols, jnp.full((pad,), col_fill, jnp.int32)])
    rp = jnp.concatenate([rows, jnp.full((pad,), n, jnp.int32)])
    cp = cp.reshape(NS * ng, G, BLK)
    rp = rp.reshape(NS * ng, G, BLK)
    if bias:
      return jnp.stack([jnp.stack([cp, rp], axis=1),
                        jnp.stack([cp + n_acc, rp], axis=1)])
    return jnp.stack([cp, rp], axis=1)

  idxb = prep(edge_index_b[1], edge_index_b[0], 0, True)
  idxa = prep(edge_index_a[1], edge_index_a[0], 0, False)

  zeros = jnp.zeros((n_acc // NS, half), jnp.float32)
  dummy = jnp.zeros((G, BLK, half), jnp.float32)

  mesh = plsc.VectorSubcoreMesh(core_axis_name="c", subcore_axis_name="s")
  fn = pl.kernel(
      functools.partial(_sc_body, n_acc, ng),
      out_type=jax.ShapeDtypeStruct((2 * n_acc, half), jnp.float32),
      mesh=mesh,
      scratch_types=[
          pltpu.VMEM_SHARED((n_acc, half), jnp.float32),
          pltpu.VMEM_SHARED((n_acc, half), jnp.float32),
          pltpu.VMEM((2, G, BLK), jnp.int32),
          pltpu.VMEM((2, G, BLK), jnp.int32),
          pltpu.VMEM((G, BLK, half), jnp.float32),
          pltpu.VMEM((G, BLK, half), jnp.float32),
          pltpu.SemaphoreType.DMA,
          pltpu.SemaphoreType.DMA,
          pltpu.SemaphoreType.DMA,
          pltpu.SemaphoreType.DMA,
      ],
      compiler_params=pltpu.CompilerParams(use_tc_tiling_on_sc=False),
  )
  outflat = fn(xflat, idxb, idxa, zeros, dummy)
  return jnp.concatenate([outflat[:n], outflat[n_acc:n_acc + n]], axis=1)
